# baseline (speedup 1.0000x reference)
import functools

import jax
import jax.numpy as jnp
from jax import lax
from jax.experimental import pallas as pl
from jax.experimental.pallas import tpu as pltpu
from jax.experimental.pallas import tpu_sc as plsc

BATCH = 16384
KNOW = 32
NC = 2
NS = 16
NW = NC * NS
BPW = BATCH // NW
CHUNK = 128
NCHUNK = BPW // CHUNK


def _sigmoid(x):
    return 1.0 / (1.0 + jnp.exp(-x))


@functools.partial(
    pl.kernel,
    mesh=plsc.VectorSubcoreMesh(core_axis_name="c", subcore_axis_name="s"),
    out_type=jax.ShapeDtypeStruct((BATCH,), jnp.float32),
    compiler_params=pltpu.CompilerParams(
        needs_layout_passes=False, use_tc_tiling_on_sc=False),
    scratch_types=[
        pltpu.VMEM((NCHUNK, CHUNK), jnp.int32),
        pltpu.VMEM((NCHUNK, CHUNK), jnp.int32),
        pltpu.VMEM((BPW, KNOW), jnp.float32),
        pltpu.VMEM((BPW, KNOW), jnp.float32),
        pltpu.VMEM((BPW,), jnp.float32),
        pltpu.VMEM((BPW, KNOW), jnp.float32),
        pltpu.VMEM((BPW,), jnp.float32),
        pltpu.SemaphoreType.DMA,
    ],
)
def _sc_compute_if(sid_hbm, q_hbm, qline_hbm, stud_hbm, diff_hbm, disc_hbm,
                   out_hbm, sid_v, qid_v, prof_v, dif_v, disc_v, qline_v,
                   out_v, sem):
    wid = lax.axis_index("s") * NC + lax.axis_index("c")
    base = wid * BPW

    pltpu.sync_copy(sid_hbm.at[pl.ds(wid * NCHUNK, NCHUNK)], sid_v)
    pltpu.sync_copy(q_hbm.at[pl.ds(wid * NCHUNK, NCHUNK)], qid_v)

    copies = []
    for j in range(NCHUNK):
        dst = pl.ds(j * CHUNK, CHUNK)
        copies.append(pltpu.make_async_copy(
            stud_hbm.at[sid_v.at[j]], prof_v.at[dst], sem))
        copies.append(pltpu.make_async_copy(
            diff_hbm.at[qid_v.at[j]], dif_v.at[dst], sem))
        copies.append(pltpu.make_async_copy(
            disc_hbm.at[qid_v.at[j]], disc_v.at[pl.ds(j * CHUNK, CHUNK)], sem))
    copies.append(pltpu.make_async_copy(
        qline_hbm.at[pl.ds(base, BPW)], qline_v, sem))
    for c in copies:
        c.start()
    for c in copies:
        c.wait()

    lanes = lax.iota(jnp.int32, 16)

    def block_body(b, _):
        acc = jnp.zeros((16,), jnp.float32)
        for j in range(16):
            i = b * 16 + j
            p0 = prof_v[i, pl.ds(0, 16)]
            p1 = prof_v[i, pl.ds(16, 16)]
            d0 = dif_v[i, pl.ds(0, 16)]
            d1 = dif_v[i, pl.ds(16, 16)]
            q0 = qline_v[i, pl.ds(0, 16)]
            q1 = qline_v[i, pl.ds(16, 16)]
            f = (q0 * (_sigmoid(p0) - _sigmoid(d0))
                 + q1 * (_sigmoid(p1) - _sigmoid(d1)))
            acc = jnp.where(lanes == j, jnp.sum(f), acc)
        disc16 = disc_v[pl.ds(b * 16, 16)]
        out_v[pl.ds(b * 16, 16)] = _sigmoid(_sigmoid(disc16) * acc)
        return 0

    lax.fori_loop(0, BPW // 16, block_body, 0)

    pltpu.sync_copy(out_v, out_hbm.at[pl.ds(base, BPW)])


def kernel(student_id, question, q_matrix_line, student_emb_w, difficulty_w,
           discrimination_w):
    sid2 = student_id.astype(jnp.int32).reshape(BATCH // CHUNK, CHUNK)
    q2 = question.astype(jnp.int32).reshape(BATCH // CHUNK, CHUNK)
    return _sc_compute_if(sid2, q2, q_matrix_line, student_emb_w,
                          difficulty_w, discrimination_w.reshape(-1))
